# baseline (speedup 1.0000x reference)
import functools

import jax
import jax.numpy as jnp
from jax import lax
from jax.experimental import pallas as pl
from jax.experimental.pallas import tpu as pltpu
from jax.experimental.pallas import tpu_sc as plsc

_D = 768
_LANES = 16
_NC = 2
_NS = 16
_NW = _NC * _NS
_NBUF = 2


@functools.lru_cache(maxsize=None)
def _emb_kernel(n_rows: int, rows_pw: int, chunk: int, n_type: int,
                n_pos: int):
    mesh = plsc.VectorSubcoreMesh(core_axis_name="c", subcore_axis_name="s")
    n_chunks = rows_pw // chunk
    n_slices = _D // _LANES
    pos_share = n_pos // _NS
    assert n_chunks % _NBUF == 0

    @functools.partial(
        pl.kernel, mesh=mesh,
        out_type=jax.ShapeDtypeStruct((n_rows, _D), jnp.float32),
        scratch_types=[
            pltpu.VMEM((n_chunks, chunk), jnp.int32),
            pltpu.VMEM((n_chunks, chunk), jnp.int32),
            pltpu.VMEM((rows_pw + _LANES,), jnp.int32),
            pltpu.VMEM((n_type, _D), jnp.float32),
        ] + [pltpu.VMEM((chunk, _D), jnp.float32),
             pltpu.VMEM((chunk, _D // 2), jnp.int32),
             pltpu.VMEM((chunk, _D), jnp.float32)] * _NBUF + [
            pltpu.SemaphoreType.DMA,
            pltpu.SemaphoreType.DMA,
            pltpu.SemaphoreType.DMA,
            pltpu.SemaphoreType.DMA,
        ],
    )
    def body(iw_hbm, ip_hbm, it_hbm, wt_hbm, pt_hbm, tt_hbm, out_hbm,
             iw_v, ip_v, it_v, tt_v,
             w0, p0, r0, w1, p1, r1,
             g0, g1, o0, o1):
        w_v, p_v, r_v = (w0, w1), (p0, p1), (r0, r1)
        gsem, osem = (g0, g1), (o0, o1)
        sid = lax.axis_index("s")
        wid = sid * _NC + lax.axis_index("c")
        base = wid * rows_pw
        pltpu.sync_copy(iw_hbm.at[wid], iw_v)
        pltpu.sync_copy(ip_hbm.at[wid], ip_v)

        def fire_gathers(k, b):
            pltpu.async_copy(wt_hbm.at[iw_v.at[k]], w_v[b], gsem[b])
            pltpu.async_copy(pt_hbm.at[ip_v.at[k]], p_v[b], gsem[b])

        def wait_gathers(k, b):
            pltpu.make_async_copy(wt_hbm.at[iw_v.at[k]],
                                  w_v[b], gsem[b]).wait()
            pltpu.make_async_copy(pt_hbm.at[ip_v.at[k]],
                                  p_v[b], gsem[b]).wait()

        def wait_out(k, b):
            off = k * chunk
            pltpu.make_async_copy(r_v[b], out_hbm.at[pl.ds(base + off, chunk)],
                                  osem[b]).wait()

        for b in range(_NBUF):
            fire_gathers(b, b)
        pltpu.sync_copy(it_hbm.at[pl.ds(base, rows_pw)],
                        it_v.at[pl.ds(0, rows_pw)])
        pltpu.sync_copy(tt_hbm, tt_v)

        def do_group(g, carry):
            for b in range(_NBUF):
                k = g * _NBUF + b
                wait_gathers(k, b)

                @pl.when(g >= 1)
                def _():
                    wait_out(k - _NBUF, b)

                tg = it_v[pl.ds(k * chunk, _LANES)].astype(jnp.float32)
                facs = []
                for r in range(chunk):
                    facs.append(jnp.full((_LANES,), tg[r], jnp.float32))

                def do_slice(j, carry2):
                    s0 = pl.ds(j * 2 * _LANES, _LANES)
                    s1 = pl.ds(j * 2 * _LANES + _LANES, _LANES)
                    t0a = tt_v[0, s0]
                    dta = tt_v[1, s0] - t0a
                    t0b = tt_v[0, s1]
                    dtb = tt_v[1, s1] - t0b
                    for r in range(chunk):
                        wrd = p_v[b][r, pl.ds(j * _LANES, _LANES)]
                        plo = lax.bitcast_convert_type(wrd << 16,
                                                       jnp.float32)
                        phi = lax.bitcast_convert_type(
                            wrd & jnp.int32(-65536), jnp.float32)
                        r_v[b][r, s0] = (w_v[b][r, s0] + plo
                                         + (t0a + facs[r] * dta))
                        r_v[b][r, s1] = (w_v[b][r, s1] + phi
                                         + (t0b + facs[r] * dtb))
                    return carry2

                lax.fori_loop(0, n_slices // 2, do_slice, 0)
                pltpu.async_copy(r_v[b], out_hbm.at[pl.ds(base + k * chunk, chunk)],
                                 osem[b])

                @pl.when(k + _NBUF < n_chunks)
                def _():
                    fire_gathers(k + _NBUF, b)
            return carry

        lax.fori_loop(0, n_chunks // _NBUF, do_group, 0)
        for b in range(_NBUF):
            wait_out(n_chunks - _NBUF + b, b)

    return body


def kernel(input_ids, position_ids, token_type_ids, word_embeddings,
           position_embeddings, token_type_embeddings):
    b, s = input_ids.shape
    n_rows = b * s
    rows_pw = n_rows // _NW
    chunk = 16
    iw = input_ids.reshape(_NW, rows_pw // chunk, chunk).astype(jnp.int32)
    ip = position_ids.reshape(_NW, rows_pw // chunk, chunk).astype(jnp.int32)
    it = token_type_ids.reshape(n_rows).astype(jnp.int32)
    cols = jnp.arange(_D)
    perm = (cols // 32) * 32 + (cols % 2) * 16 + (cols % 32) // 2
    ptb = position_embeddings.astype(jnp.bfloat16)[:, perm]
    position_embeddings = jax.lax.bitcast_convert_type(
        ptb.reshape(ptb.shape[0], _D // 2, 2), jnp.int32)
    n_type = token_type_embeddings.shape[0]
    assert n_type == 2, "kernel specialized for a 2-row token-type table"
    n_pos = position_embeddings.shape[0]
    k = _emb_kernel(n_rows, rows_pw, chunk=16, n_type=n_type, n_pos=n_pos)
    out = k(iw, ip, it, word_embeddings, position_embeddings,
            token_type_embeddings)
    return out.reshape(b, s, _D)


# speedup vs baseline: 1.2978x; 1.2978x over previous
import functools

import jax
import jax.numpy as jnp
from jax import lax
from jax.experimental import pallas as pl
from jax.experimental.pallas import tpu as pltpu
from jax.experimental.pallas import tpu_sc as plsc

_D = 768
_LANES = 16
_NC = 2
_NS = 16
_NW = _NC * _NS
_NBUF = 2


@functools.lru_cache(maxsize=None)
def _emb_kernel(n_rows: int, rows_pw: int, chunk: int, n_type: int,
                n_pos: int):
    mesh = plsc.VectorSubcoreMesh(core_axis_name="c", subcore_axis_name="s")
    n_chunks = rows_pw // chunk
    n_slices = _D // _LANES
    pos_share = n_pos // _NS
    assert n_chunks % _NBUF == 0

    @functools.partial(
        pl.kernel, mesh=mesh,
        out_type=jax.ShapeDtypeStruct((n_rows, _D), jnp.float32),
        scratch_types=[
            pltpu.VMEM((n_chunks, chunk), jnp.int32),
            pltpu.VMEM((n_chunks, chunk), jnp.int32),
            pltpu.VMEM((rows_pw + _LANES,), jnp.int32),
            pltpu.VMEM((n_type, _D), jnp.float32),
        ] + [pltpu.VMEM((chunk, _D), jnp.float32),
             pltpu.VMEM((chunk, _D // 2), jnp.int32),
             pltpu.VMEM((chunk, _D), jnp.float32)] * _NBUF + [
            pltpu.SemaphoreType.DMA,
            pltpu.SemaphoreType.DMA,
            pltpu.SemaphoreType.DMA,
            pltpu.SemaphoreType.DMA,
        ],
    )
    def body(iw_hbm, ip_hbm, it_hbm, wt_hbm, pt_hbm, tt_hbm, out_hbm,
             iw_v, ip_v, it_v, tt_v,
             w0, p0, r0, w1, p1, r1,
             g0, g1, o0, o1):
        w_v, p_v, r_v = (w0, w1), (p0, p1), (r0, r1)
        gsem, osem = (g0, g1), (o0, o1)
        sid = lax.axis_index("s")
        wid = sid * _NC + lax.axis_index("c")
        base = wid * rows_pw
        pltpu.sync_copy(iw_hbm.at[wid], iw_v)
        pltpu.sync_copy(ip_hbm.at[wid], ip_v)

        def fire_gathers(k, b):
            pltpu.async_copy(wt_hbm.at[iw_v.at[k]], w_v[b], gsem[b])
            pltpu.async_copy(pt_hbm.at[ip_v.at[k]], p_v[b], gsem[b])

        def wait_gathers(k, b):
            pltpu.make_async_copy(wt_hbm.at[iw_v.at[k]],
                                  w_v[b], gsem[b]).wait()
            pltpu.make_async_copy(pt_hbm.at[ip_v.at[k]],
                                  p_v[b], gsem[b]).wait()

        def wait_out(k, b):
            off = k * chunk
            pltpu.make_async_copy(r_v[b], out_hbm.at[pl.ds(base + off, chunk)],
                                  osem[b]).wait()

        for b in range(_NBUF):
            fire_gathers(b, b)
        pltpu.sync_copy(it_hbm.at[pl.ds(base, rows_pw)],
                        it_v.at[pl.ds(0, rows_pw)])
        pltpu.sync_copy(tt_hbm, tt_v)

        def do_group(g, carry):
            for b in range(_NBUF):
                k = g * _NBUF + b
                wait_gathers(k, b)

                @pl.when(g >= 1)
                def _():
                    wait_out(k - _NBUF, b)

                tg = it_v[pl.ds(k * chunk, _LANES)].astype(jnp.float32)
                facs = []
                for r in range(chunk):
                    facs.append(jnp.full((_LANES,), tg[r], jnp.float32))

                def do_slice(j, carry2):
                    s0 = pl.ds(j * 2 * _LANES, _LANES)
                    s1 = pl.ds(j * 2 * _LANES + _LANES, _LANES)
                    t0a = tt_v[0, s0]
                    dta = tt_v[1, s0] - t0a
                    t0b = tt_v[0, s1]
                    dtb = tt_v[1, s1] - t0b
                    for r in range(chunk):
                        wrd = p_v[b][r, pl.ds(j * _LANES, _LANES)]
                        plo = lax.bitcast_convert_type(wrd << 16,
                                                       jnp.float32)
                        phi = lax.bitcast_convert_type(
                            wrd & jnp.int32(-65536), jnp.float32)
                        r_v[b][r, s0] = (w_v[b][r, s0] + plo
                                         + (t0a + facs[r] * dta))
                        r_v[b][r, s1] = (w_v[b][r, s1] + phi
                                         + (t0b + facs[r] * dtb))
                    return carry2

                lax.fori_loop(0, n_slices // 2, do_slice, 0)
                pltpu.async_copy(r_v[b], out_hbm.at[pl.ds(base + k * chunk, chunk)],
                                 osem[b])

                @pl.when(k + _NBUF < n_chunks)
                def _():
                    fire_gathers(k + _NBUF, b)
            return carry

        lax.fori_loop(0, n_chunks // _NBUF, do_group, 0)
        for b in range(_NBUF):
            wait_out(n_chunks - _NBUF + b, b)

    return body


def kernel(input_ids, position_ids, token_type_ids, word_embeddings,
           position_embeddings, token_type_embeddings):
    b, s = input_ids.shape
    n_rows = b * s
    rows_pw = n_rows // _NW
    chunk = 16
    iw = input_ids.reshape(_NW, rows_pw // chunk, chunk).astype(jnp.int32)
    ip = position_ids.reshape(_NW, rows_pw // chunk, chunk).astype(jnp.int32)
    it = token_type_ids.reshape(n_rows).astype(jnp.int32)
    n_p = position_embeddings.shape[0]
    ptb = (position_embeddings.astype(jnp.bfloat16)
           .reshape(n_p, _D // 32, 2, 16).swapaxes(-1, -2))
    position_embeddings = jax.lax.bitcast_convert_type(
        ptb.reshape(ptb.shape[0], _D // 2, 2), jnp.int32)
    n_type = token_type_embeddings.shape[0]
    assert n_type == 2, "kernel specialized for a 2-row token-type table"
    n_pos = position_embeddings.shape[0]
    k = _emb_kernel(n_rows, rows_pw, chunk=16, n_type=n_type, n_pos=n_pos)
    out = k(iw, ip, it, word_embeddings, position_embeddings,
            token_type_embeddings)
    return out.reshape(b, s, _D)


# speedup vs baseline: 2.5279x; 1.9479x over previous
import functools

import jax
import jax.numpy as jnp
from jax import lax
from jax.experimental import pallas as pl
from jax.experimental.pallas import tpu as pltpu
from jax.experimental.pallas import tpu_sc as plsc

_D = 768
_LANES = 16
_NC = 2
_NS = 16
_NW = _NC * _NS
_NBUF = 2


@functools.lru_cache(maxsize=None)
def _emb_kernel(n_rows: int, rows_pw: int, chunk: int, n_type: int,
                n_pos: int):
    mesh = plsc.VectorSubcoreMesh(core_axis_name="c", subcore_axis_name="s")
    n_chunks = rows_pw // chunk
    n_slices = _D // _LANES
    pos_share = n_pos // _NS
    assert n_chunks % _NBUF == 0

    @functools.partial(
        pl.kernel, mesh=mesh,
        out_type=jax.ShapeDtypeStruct((n_rows, _D), jnp.float32),
        scratch_types=[
            pltpu.VMEM((n_chunks, chunk), jnp.int32),
            pltpu.VMEM((n_chunks, chunk), jnp.int32),
            pltpu.VMEM((rows_pw + _LANES,), jnp.int32),
            pltpu.VMEM((n_type, _D), jnp.float32),
        ] + [pltpu.VMEM((chunk, _D), jnp.float32)] * (3 * _NBUF) + [
            pltpu.SemaphoreType.DMA,
            pltpu.SemaphoreType.DMA,
            pltpu.SemaphoreType.DMA,
            pltpu.SemaphoreType.DMA,
        ],
    )
    def body(iw_hbm, ip_hbm, it_hbm, wt_hbm, pt_hbm, tt_hbm, out_hbm,
             iw_v, ip_v, it_v, tt_v,
             w0, p0, r0, w1, p1, r1,
             g0, g1, o0, o1):
        w_v, p_v, r_v = (w0, w1), (p0, p1), (r0, r1)
        gsem, osem = (g0, g1), (o0, o1)
        sid = lax.axis_index("s")
        wid = sid * _NC + lax.axis_index("c")
        base = wid * rows_pw
        pltpu.sync_copy(iw_hbm.at[wid], iw_v)
        pltpu.sync_copy(ip_hbm.at[wid], ip_v)

        def fire_gathers(k, b):
            pltpu.async_copy(wt_hbm.at[iw_v.at[k]], w_v[b], gsem[b])
            pltpu.async_copy(pt_hbm.at[ip_v.at[k]], p_v[b], gsem[b])

        def wait_gathers(k, b):
            pltpu.make_async_copy(wt_hbm.at[iw_v.at[k]],
                                  w_v[b], gsem[b]).wait()
            pltpu.make_async_copy(pt_hbm.at[ip_v.at[k]],
                                  p_v[b], gsem[b]).wait()

        def wait_out(k, b):
            off = k * chunk
            pltpu.make_async_copy(r_v[b], out_hbm.at[pl.ds(base + off, chunk)],
                                  osem[b]).wait()

        for b in range(_NBUF):
            fire_gathers(b, b)
        pltpu.sync_copy(it_hbm.at[pl.ds(base, rows_pw)],
                        it_v.at[pl.ds(0, rows_pw)])
        pltpu.sync_copy(tt_hbm, tt_v)

        def do_group(g, carry):
            for b in range(_NBUF):
                k = g * _NBUF + b
                wait_gathers(k, b)

                @pl.when(g >= 1)
                def _():
                    wait_out(k - _NBUF, b)

                tg = it_v[pl.ds(k * chunk, _LANES)].astype(jnp.float32)
                facs = []
                for r in range(chunk):
                    facs.append(jnp.full((_LANES,), tg[r], jnp.float32))

                def do_slice(j, carry2):
                    s = pl.ds(j * _LANES, _LANES)
                    t0 = tt_v[0, s]
                    dt = tt_v[1, s] - t0
                    for r in range(chunk):
                        r_v[b][r, s] = (w_v[b][r, s] + p_v[b][r, s]
                                        + (t0 + facs[r] * dt))
                    return carry2

                lax.fori_loop(0, n_slices, do_slice, 0)
                pltpu.async_copy(r_v[b], out_hbm.at[pl.ds(base + k * chunk, chunk)],
                                 osem[b])

                @pl.when(k + _NBUF < n_chunks)
                def _():
                    fire_gathers(k + _NBUF, b)
            return carry

        lax.fori_loop(0, n_chunks // _NBUF, do_group, 0)
        for b in range(_NBUF):
            wait_out(n_chunks - _NBUF + b, b)

    return body


def kernel(input_ids, position_ids, token_type_ids, word_embeddings,
           position_embeddings, token_type_embeddings):
    b, s = input_ids.shape
    n_rows = b * s
    rows_pw = n_rows // _NW
    chunk = 16
    iw = input_ids.reshape(_NW, rows_pw // chunk, chunk).astype(jnp.int32)
    ip = position_ids.reshape(_NW, rows_pw // chunk, chunk).astype(jnp.int32)
    it = token_type_ids.reshape(n_rows).astype(jnp.int32)
    n_type = token_type_embeddings.shape[0]
    assert n_type == 2, "kernel specialized for a 2-row token-type table"
    n_pos = position_embeddings.shape[0]
    k = _emb_kernel(n_rows, rows_pw, chunk=16, n_type=n_type, n_pos=n_pos)
    out = k(iw, ip, it, word_embeddings, position_embeddings,
            token_type_embeddings)
    return out.reshape(b, s, _D)
